# Initial kernel scaffold; baseline (speedup 1.0000x reference)
import jax
import jax.numpy as jnp
from jax.experimental import pallas as pl


def kernel(x, gate_W, gate_b, W1, B1, W2, B2, W3, B3):
    raise NotImplementedError("write your pallas kernel here")


# speedup vs baseline: 1.2681x; 1.2681x over previous
import jax
import jax.numpy as jnp
from jax.experimental import pallas as pl
from jax.experimental.pallas import tpu as pltpu

_DIM = 768
_HID = 1536
_E = 8
_N = 2048
_TN = 256
_NT = _N // _TN


def _gate_body(x_ref, gw_ref, gb_ref, cw_ref, bal_ref):
    x = x_ref[...]
    gb = gb_ref[...]
    logits = jnp.dot(x, gw_ref[...],
                     preferred_element_type=jnp.float32) + gb
    m = jnp.max(logits, axis=-1, keepdims=True)
    ex = jnp.exp(logits - m)
    s = ex / jnp.sum(ex, axis=-1, keepdims=True)
    lane = jax.lax.broadcasted_iota(jnp.int32, s.shape, 1)
    sr = jnp.where(lane >= 1, s, -1.0)
    m1 = jnp.max(sr, axis=-1, keepdims=True)
    i1 = jnp.min(jnp.where(sr == m1, lane, 127), axis=-1, keepdims=True)
    sr2 = jnp.where(lane == i1, -1.0, sr)
    m2 = jnp.max(sr2, axis=-1, keepdims=True)
    i2 = jnp.min(jnp.where(sr2 == m2, lane, 127), axis=-1, keepdims=True)
    sel = (lane == 0) | (lane == i1) | (lane == i2)
    cw = jnp.where(sel, s, 0.0)
    cw_ref[...] = cw
    usage = jnp.sum(jnp.where(sel, 1.0, 0.0), axis=0)
    ssum = jnp.sum(cw, axis=0)
    kp = 3.0
    bal = jnp.sum(usage * ssum) * (float(_E) / (kp * _N * _N))
    bal_ref[...] = jnp.reshape(bal, (1, 1))


def _ffn_body(cw_ref, x_ref, w1_ref, b1_ref, w2_ref, b2_ref, w3_ref, b3_ref,
              out_ref, acc_ref):
    e = pl.program_id(0)
    t = pl.program_id(1)
    x = x_ref[...]
    a = jnp.dot(x, w1_ref[0], preferred_element_type=jnp.float32) + b1_ref[0]
    g = jnp.dot(x, w3_ref[0], preferred_element_type=jnp.float32) + b3_ref[0]
    h = g * jax.nn.sigmoid(g) * a
    y = jnp.dot(h, w2_ref[0], preferred_element_type=jnp.float32) + b2_ref[0]
    lane8 = jax.lax.broadcasted_iota(jnp.int32, (_TN, _E), 1)
    w = jnp.sum(jnp.where(lane8 == e, cw_ref[...], 0.0), axis=1, keepdims=True)
    contrib = w * y
    sl = pl.ds(t * _TN, _TN)

    @pl.when(e == 0)
    def _():
        acc_ref[sl, :] = contrib

    @pl.when(e > 0)
    def _():
        acc_ref[sl, :] = acc_ref[sl, :] + contrib

    @pl.when(e == _E - 1)
    def _():
        out_ref[...] = acc_ref[sl, :]


def kernel(x, gate_W, gate_b, W1, B1, W2, B2, W3, B3):
    Bb, Tt, C = x.shape
    xf = x.reshape(_N, _DIM)
    cw, bal = pl.pallas_call(
        _gate_body,
        out_shape=[
            jax.ShapeDtypeStruct((_N, _E), jnp.float32),
            jax.ShapeDtypeStruct((1, 1), jnp.float32),
        ],
    )(xf, gate_W, gate_b.reshape(1, _E))

    out = pl.pallas_call(
        _ffn_body,
        grid=(_E, _NT),
        in_specs=[
            pl.BlockSpec((_TN, _E), lambda e, t: (t, 0)),
            pl.BlockSpec((_TN, _DIM), lambda e, t: (t, 0)),
            pl.BlockSpec((1, _DIM, _HID), lambda e, t: (e, 0, 0)),
            pl.BlockSpec((1, 1, _HID), lambda e, t: (e, 0, 0)),
            pl.BlockSpec((1, _HID, _DIM), lambda e, t: (e, 0, 0)),
            pl.BlockSpec((1, 1, _DIM), lambda e, t: (e, 0, 0)),
            pl.BlockSpec((1, _DIM, _HID), lambda e, t: (e, 0, 0)),
            pl.BlockSpec((1, 1, _HID), lambda e, t: (e, 0, 0)),
        ],
        out_specs=pl.BlockSpec((_TN, _DIM), lambda e, t: (t, 0)),
        out_shape=jax.ShapeDtypeStruct((_N, _DIM), jnp.float32),
        scratch_shapes=[pltpu.VMEM((_N, _DIM), jnp.float32)],
        compiler_params=pltpu.CompilerParams(
            dimension_semantics=("arbitrary", "arbitrary")),
    )(cw, xf, W1, B1.reshape(_E, 1, _HID), W2, B2.reshape(_E, 1, _DIM),
      W3, B3.reshape(_E, 1, _HID))

    return out.reshape(Bb, Tt, C), bal.reshape(())
